# baseline (speedup 1.0000x reference)
import functools

import numpy as np
import jax
import jax.numpy as jnp
from jax import lax
from jax.experimental import pallas as pl
from jax.experimental.pallas import tpu as pltpu
from jax.experimental.pallas import tpu_sc as plsc

VOCAB = 100000
EMBED = 128
MAX_LEN = 512
B, L = 1024, 200

_NUM_CORES = 2
_NUM_SUBCORES = 16
_NW = _NUM_CORES * _NUM_SUBCORES
_LANES = 16
_CH = L
_RPW = (B * L) // _NW
_CPW = _RPW // _CH
_NPAIR = _CPW // 2
_NBUF = 2
_NIDX = 4
_G0 = 128
_G1 = _CH - _G0


def _sinusoidal_pe(max_len, d_model):
    position = np.arange(max_len, dtype=np.float64)[:, None]
    div_term = np.exp(
        np.arange(0, d_model, 2, dtype=np.float64) * -(np.log(10000.0) / d_model)
    )
    pe = np.zeros((max_len, d_model), dtype=np.float64)
    pe[:, 0::2] = np.sin(position * div_term)
    pe[:, 1::2] = np.cos(position * div_term)
    return pe.astype(np.float32)


_PE = _sinusoidal_pe(MAX_LEN, EMBED)[:L]


def _sc_body(table_hbm, idx_hbm, pe_hbm, out_hbm, idx_v, pe_v, rows, sem_i, sem_g, sem_s):
    wid = lax.axis_index("s") * _NUM_CORES + lax.axis_index("c")
    base = wid * _RPW

    pe_copy = pltpu.make_async_copy(pe_hbm, pe_v, sem_s[0])
    pe_copy.start()

    def idx_copy(c):
        s = c % _NIDX
        return pltpu.make_async_copy(
            idx_hbm.at[pl.ds(base + c * _CH, _CH)],
            idx_v.at[pl.ds(s * _CH, _CH)],
            sem_i[s],
        )

    def gathers(k):
        b = k % _NBUF
        out = []
        for j in range(2):
            c = 2 * k + j
            s = c % _NIDX
            out.append(pltpu.make_async_copy(
                table_hbm.at[idx_v.at[pl.ds(s * _CH, _G0)]],
                rows[b].at[pl.ds(j * _CH, _G0), :],
                sem_g[b],
            ))
            out.append(pltpu.make_async_copy(
                table_hbm.at[idx_v.at[pl.ds(s * _CH + _G0, _G1)]],
                rows[b].at[pl.ds(j * _CH + _G0, _G1), :],
                sem_g[b],
            ))
        return out

    def store(k):
        b = k % _NBUF
        return pltpu.make_async_copy(
            rows[b], out_hbm.at[pl.ds(base + 2 * k * _CH, 2 * _CH)], sem_s[b]
        )

    for c in range(_NIDX):
        idx_copy(c).start()
    for c in range(2):
        idx_copy(c).wait()
    for g in gathers(0):
        g.start()
    pe_copy.wait()

    for k in range(_NPAIR):
        for g in gathers(k):
            g.wait()
        b = k % _NBUF

        p = k + 1
        if p < _NPAIR:
            if p >= _NBUF:
                store(p - _NBUF).wait()
            idx_copy(2 * p).wait()
            idx_copy(2 * p + 1).wait()
            for g in gathers(p):
                g.start()
            if p + 1 < _NPAIR:
                idx_copy(2 * p + 2).start()
                idx_copy(2 * p + 3).start()

        for j in range(2):
            @pl.loop(0, _CH, unroll=4)
            def _add(r, b=b, j=j):
                for d in range(EMBED // _LANES):
                    sl = pl.ds(d * _LANES, _LANES)
                    plsc.addupdate(rows[b].at[j * _CH + r, sl], pe_v[r, sl])

        store(k).start()

    for k in range(max(0, _NPAIR - _NBUF), _NPAIR):
        store(k).wait()


@functools.partial(
    pl.kernel,
    out_type=jax.ShapeDtypeStruct((B * L, EMBED), jnp.float32),
    mesh=plsc.VectorSubcoreMesh(core_axis_name="c", subcore_axis_name="s"),
    scratch_types=[
        pltpu.VMEM((_NIDX * _CH,), jnp.int32),
        pltpu.VMEM((L, EMBED), jnp.float32),
        [pltpu.VMEM((2 * _CH, EMBED), jnp.float32) for _ in range(_NBUF)],
        [pltpu.SemaphoreType.DMA for _ in range(_NIDX)],
        [pltpu.SemaphoreType.DMA for _ in range(_NBUF)],
        [pltpu.SemaphoreType.DMA for _ in range(_NBUF)],
    ],
)
def _sc_embed(table_hbm, idx_hbm, pe_hbm, out_hbm, idx_v, pe_v, rows, sem_i, sem_g, sem_s):
    _sc_body(table_hbm, idx_hbm, pe_hbm, out_hbm, idx_v, pe_v, rows, sem_i, sem_g, sem_s)


def kernel(sequence, token_table):
    idx = sequence.reshape(-1).astype(jnp.int32)
    out = _sc_embed(token_table, idx, jnp.asarray(_PE))
    return out.reshape(B, L, EMBED)


# speedup vs baseline: 1.1318x; 1.1318x over previous
import functools

import numpy as np
import jax
import jax.numpy as jnp
from jax import lax
from jax.experimental import pallas as pl
from jax.experimental.pallas import tpu as pltpu
from jax.experimental.pallas import tpu_sc as plsc

VOCAB = 100000
EMBED = 128
MAX_LEN = 512
B, L = 1024, 200

_NUM_CORES = 2
_NUM_SUBCORES = 16
_NW = _NUM_CORES * _NUM_SUBCORES
_LANES = 16
_CH = L
_RPW = (B * L) // _NW
_CPW = _RPW // _CH
_NBUF = 4
_LOOK = 2
_G0 = 128
_G1 = _CH - _G0


def _sinusoidal_pe(max_len, d_model):
    position = np.arange(max_len, dtype=np.float64)[:, None]
    div_term = np.exp(
        np.arange(0, d_model, 2, dtype=np.float64) * -(np.log(10000.0) / d_model)
    )
    pe = np.zeros((max_len, d_model), dtype=np.float64)
    pe[:, 0::2] = np.sin(position * div_term)
    pe[:, 1::2] = np.cos(position * div_term)
    return pe.astype(np.float32)


_PE = _sinusoidal_pe(MAX_LEN, EMBED)[:L]


def _sc_body(table_hbm, idx_hbm, pe_hbm, out_hbm, idx_v, pe_v, rows, sem_i, sem_g, sem_s):
    wid = lax.axis_index("s") * _NUM_CORES + lax.axis_index("c")
    base = wid * _RPW

    pe_copy = pltpu.make_async_copy(pe_hbm, pe_v, sem_s[0])
    pe_copy.start()

    def idx_copy(c):
        b = c % _NBUF
        return pltpu.make_async_copy(
            idx_hbm.at[pl.ds(base + c * _CH, _CH)],
            idx_v.at[pl.ds(b * _CH, _CH)],
            sem_i[b],
        )

    def gathers(c):
        b = c % _NBUF
        return (
            pltpu.make_async_copy(
                table_hbm.at[idx_v.at[pl.ds(b * _CH, _G0)]],
                rows[b].at[pl.ds(0, _G0), :],
                sem_g[b],
            ),
            pltpu.make_async_copy(
                table_hbm.at[idx_v.at[pl.ds(b * _CH + _G0, _G1)]],
                rows[b].at[pl.ds(_G0, _G1), :],
                sem_g[b],
            ),
        )

    def store(c):
        b = c % _NBUF
        return pltpu.make_async_copy(
            rows[b], out_hbm.at[pl.ds(base + c * _CH, _CH)], sem_s[b]
        )

    for c in range(min(_LOOK + 1, _CPW)):
        idx_copy(c).start()
    for c in range(_LOOK):
        idx_copy(c).wait()
        g0, g1 = gathers(c)
        g0.start()
        g1.start()
    pe_copy.wait()

    for c in range(_CPW):
        g0, g1 = gathers(c)
        g0.wait()
        g1.wait()
        b = c % _NBUF

        p = c + _LOOK
        if p < _CPW:
            if p >= _NBUF:
                store(p - _NBUF).wait()
            idx_copy(p).wait()
            n0, n1 = gathers(p)
            n0.start()
            n1.start()
            if p + 1 < _CPW:
                idx_copy(p + 1).start()

        @pl.loop(0, _CH, unroll=4)
        def _add(r, b=b):
            for d in range(EMBED // _LANES):
                sl = pl.ds(d * _LANES, _LANES)
                plsc.addupdate(rows[b].at[r, sl], pe_v[r, sl])

        store(c).start()

    for c in range(max(0, _CPW - _NBUF), _CPW):
        store(c).wait()


@functools.partial(
    pl.kernel,
    out_type=jax.ShapeDtypeStruct((B * L, EMBED), jnp.float32),
    mesh=plsc.VectorSubcoreMesh(core_axis_name="c", subcore_axis_name="s"),
    scratch_types=[
        pltpu.VMEM((_NBUF * _CH,), jnp.int32),
        pltpu.VMEM((L, EMBED), jnp.float32),
        [pltpu.VMEM((_CH, EMBED), jnp.float32) for _ in range(_NBUF)],
        [pltpu.SemaphoreType.DMA for _ in range(_NBUF)],
        [pltpu.SemaphoreType.DMA for _ in range(_NBUF)],
        [pltpu.SemaphoreType.DMA for _ in range(_NBUF)],
    ],
)
def _sc_embed(table_hbm, idx_hbm, pe_hbm, out_hbm, idx_v, pe_v, rows, sem_i, sem_g, sem_s):
    _sc_body(table_hbm, idx_hbm, pe_hbm, out_hbm, idx_v, pe_v, rows, sem_i, sem_g, sem_s)


def kernel(sequence, token_table):
    idx = sequence.reshape(-1).astype(jnp.int32)
    out = _sc_embed(token_table, idx, jnp.asarray(_PE))
    return out.reshape(B, L, EMBED)
